# Initial kernel scaffold; baseline (speedup 1.0000x reference)
import jax
import jax.numpy as jnp
from jax.experimental import pallas as pl


def kernel(x, weight):
    raise NotImplementedError("write your pallas kernel here")


# speedup vs baseline: 1.1067x; 1.1067x over previous
import functools

import jax
import jax.numpy as jnp
from jax.experimental import pallas as pl

N_EXPERTS = 64
TOPK = 8
N_GROUPS = 8
GROUP_SIZE = N_EXPERTS // N_GROUPS
TOPK_GROUPS = 4
ROUTE_SCALE = 2.5

NEG_INF = float("-inf")


def _gate_kernel(x_ref, w_ref, wout_ref, iout_ref):
    x = x_ref[...]
    w = w_ref[...]
    scores = jax.lax.dot_general(
        x, w, (((1,), (1,)), ((), ())), preferred_element_type=jnp.float32
    )
    scores = jax.nn.sigmoid(scores)
    t = scores.shape[0]

    lane = jax.lax.broadcasted_iota(jnp.int32, (t, N_EXPERTS), 1)
    glane = lane // GROUP_SIZE

    gmax = jnp.concatenate(
        [
            jnp.broadcast_to(
                jnp.max(
                    scores[:, g * GROUP_SIZE : (g + 1) * GROUP_SIZE],
                    axis=1,
                    keepdims=True,
                ),
                (t, GROUP_SIZE),
            )
            for g in range(N_GROUPS)
        ],
        axis=1,
    )

    work = gmax
    sel = jnp.zeros((t, N_EXPERTS), jnp.bool_)
    for _ in range(TOPK_GROUPS):
        m = jnp.max(work, axis=1, keepdims=True)
        cand = jnp.where(work == m, glane, N_GROUPS)
        best_g = jnp.min(cand, axis=1, keepdims=True)
        pick = glane == best_g
        sel = jnp.logical_or(sel, pick)
        work = jnp.where(pick, NEG_INF, work)

    masked = jnp.where(sel, scores, NEG_INF)
    w_cols = []
    i_cols = []
    for _ in range(TOPK):
        m = jnp.max(masked, axis=1, keepdims=True)
        cand = jnp.where(masked == m, lane, N_EXPERTS)
        best = jnp.min(cand, axis=1, keepdims=True)
        w_cols.append(m)
        i_cols.append(best)
        masked = jnp.where(lane == best, NEG_INF, masked)
    wts = jnp.concatenate(w_cols, axis=1)
    idx = jnp.concatenate(i_cols, axis=1)
    wts = wts / jnp.sum(wts, axis=1, keepdims=True) * ROUTE_SCALE

    wout_ref[...] = wts
    iout_ref[...] = idx


@functools.partial(jax.jit, static_argnames=())
def kernel(x, weight):
    tokens, dim = x.shape
    tile_t = min(512, tokens)
    grid = (tokens // tile_t,)
    wts, idx = pl.pallas_call(
        _gate_kernel,
        grid=grid,
        in_specs=[
            pl.BlockSpec((tile_t, dim), lambda i: (i, 0)),
            pl.BlockSpec((N_EXPERTS, dim), lambda i: (0, 0)),
        ],
        out_specs=[
            pl.BlockSpec((tile_t, TOPK), lambda i: (i, 0)),
            pl.BlockSpec((tile_t, TOPK), lambda i: (i, 0)),
        ],
        out_shape=[
            jax.ShapeDtypeStruct((tokens, TOPK), jnp.float32),
            jax.ShapeDtypeStruct((tokens, TOPK), jnp.int32),
        ],
    )(x.astype(jnp.float32), weight.astype(jnp.float32))
    return wts, idx


# speedup vs baseline: 2.2284x; 2.0136x over previous
import functools

import jax
import jax.numpy as jnp
from jax.experimental import pallas as pl

N_EXPERTS = 64
TOPK = 8
N_GROUPS = 8
GROUP_SIZE = N_EXPERTS // N_GROUPS
TOPK_GROUPS = 4
ROUTE_SCALE = 2.5

NEG_INF = float("-inf")


def _gate_kernel(x_ref, w_ref, wout_ref, iout_ref):
    x = x_ref[...]
    w = w_ref[...]
    scores = jax.lax.dot_general(
        w, x, (((1,), (1,)), ((), ())), preferred_element_type=jnp.float32
    )
    scores = jax.nn.sigmoid(scores)
    t = scores.shape[1]

    erow = jax.lax.broadcasted_iota(jnp.int32, (N_EXPERTS, t), 0)
    grow8 = jax.lax.broadcasted_iota(jnp.int32, (N_GROUPS, t), 0)

    gmax = jnp.max(scores.reshape(N_GROUPS, GROUP_SIZE, t), axis=1)

    work = gmax
    sel8 = jnp.zeros((N_GROUPS, t), jnp.bool_)
    for _ in range(TOPK_GROUPS):
        m = jnp.max(work, axis=0, keepdims=True)
        cand = jnp.where(work == m, grow8, N_GROUPS)
        best_g = jnp.min(cand, axis=0, keepdims=True)
        pick = grow8 == best_g
        sel8 = jnp.logical_or(sel8, pick)
        work = jnp.where(pick, NEG_INF, work)

    sel = jnp.broadcast_to(sel8[:, None, :], (N_GROUPS, GROUP_SIZE, t)).reshape(
        N_EXPERTS, t
    )
    masked = jnp.where(sel, scores, NEG_INF)
    w_rows = []
    i_rows = []
    for _ in range(TOPK):
        m = jnp.max(masked, axis=0, keepdims=True)
        cand = jnp.where(masked == m, erow, N_EXPERTS)
        best = jnp.min(cand, axis=0, keepdims=True)
        w_rows.append(m)
        i_rows.append(best)
        masked = jnp.where(erow == best, NEG_INF, masked)
    wts = jnp.concatenate(w_rows, axis=0)
    idx = jnp.concatenate(i_rows, axis=0)
    wts = wts / jnp.sum(wts, axis=0, keepdims=True) * ROUTE_SCALE

    wout_ref[...] = wts.T
    iout_ref[...] = idx.T


@functools.partial(jax.jit, static_argnames=())
def kernel(x, weight):
    tokens, dim = x.shape
    tile_t = min(512, tokens)
    grid = (tokens // tile_t,)
    wts, idx = pl.pallas_call(
        _gate_kernel,
        grid=grid,
        in_specs=[
            pl.BlockSpec((tile_t, dim), lambda i: (i, 0)),
            pl.BlockSpec((N_EXPERTS, dim), lambda i: (0, 0)),
        ],
        out_specs=[
            pl.BlockSpec((tile_t, TOPK), lambda i: (i, 0)),
            pl.BlockSpec((tile_t, TOPK), lambda i: (i, 0)),
        ],
        out_shape=[
            jax.ShapeDtypeStruct((tokens, TOPK), jnp.float32),
            jax.ShapeDtypeStruct((tokens, TOPK), jnp.int32),
        ],
    )(x.astype(jnp.float32), weight.astype(jnp.float32))
    return wts, idx


# speedup vs baseline: 2.5881x; 1.1614x over previous
import functools

import jax
import jax.numpy as jnp
from jax.experimental import pallas as pl

N_EXPERTS = 64
TOPK = 8
N_GROUPS = 8
GROUP_SIZE = N_EXPERTS // N_GROUPS
TOPK_GROUPS = 4
ROUTE_SCALE = 2.5

NEG_INF = float("-inf")


def _gate_kernel(x_ref, w_ref, wout_ref, iout_ref):
    x = x_ref[...]
    w = w_ref[...]
    scores = jax.lax.dot_general(
        w, x, (((1,), (1,)), ((), ())), preferred_element_type=jnp.float32
    )
    scores = jax.nn.sigmoid(scores)
    t = scores.shape[1]

    erow = jax.lax.broadcasted_iota(jnp.int32, (N_EXPERTS, t), 0)
    grow8 = jax.lax.broadcasted_iota(jnp.int32, (N_GROUPS, t), 0)

    gmax = jnp.max(scores.reshape(N_GROUPS, GROUP_SIZE, t), axis=1)

    work = gmax
    sel8 = jnp.zeros((N_GROUPS, t), jnp.bool_)
    for _ in range(TOPK_GROUPS):
        m = jnp.max(work, axis=0, keepdims=True)
        cand = jnp.where(work == m, grow8, N_GROUPS)
        best_g = jnp.min(cand, axis=0, keepdims=True)
        pick = grow8 == best_g
        sel8 = jnp.logical_or(sel8, pick)
        work = jnp.where(pick, NEG_INF, work)

    sel = jnp.broadcast_to(sel8[:, None, :], (N_GROUPS, GROUP_SIZE, t)).reshape(
        N_EXPERTS, t
    )
    masked = jnp.where(sel, scores, NEG_INF)
    w_rows = []
    i_rows = []
    for _ in range(TOPK):
        m = jnp.max(masked, axis=0, keepdims=True)
        cand = jnp.where(masked == m, erow, N_EXPERTS)
        best = jnp.min(cand, axis=0, keepdims=True)
        w_rows.append(m)
        i_rows.append(best)
        masked = jnp.where(erow == best, NEG_INF, masked)
    wts = jnp.concatenate(w_rows, axis=0)
    idx = jnp.concatenate(i_rows, axis=0)
    wts = wts / jnp.sum(wts, axis=0, keepdims=True) * ROUTE_SCALE

    wout_ref[...] = wts.T
    iout_ref[...] = idx.T


@functools.partial(jax.jit, static_argnames=())
def kernel(x, weight):
    tokens, dim = x.shape
    tile_t = min(1024, tokens)
    grid = (tokens // tile_t,)
    wts, idx = pl.pallas_call(
        _gate_kernel,
        grid=grid,
        in_specs=[
            pl.BlockSpec((tile_t, dim), lambda i: (i, 0)),
            pl.BlockSpec((N_EXPERTS, dim), lambda i: (0, 0)),
        ],
        out_specs=[
            pl.BlockSpec((tile_t, TOPK), lambda i: (i, 0)),
            pl.BlockSpec((tile_t, TOPK), lambda i: (i, 0)),
        ],
        out_shape=[
            jax.ShapeDtypeStruct((tokens, TOPK), jnp.float32),
            jax.ShapeDtypeStruct((tokens, TOPK), jnp.int32),
        ],
    )(x.astype(jnp.float32), weight.astype(jnp.float32))
    return wts, idx


# speedup vs baseline: 2.7489x; 1.0621x over previous
import functools

import jax
import jax.numpy as jnp
from jax.experimental import pallas as pl

N_EXPERTS = 64
TOPK = 8
N_GROUPS = 8
GROUP_SIZE = N_EXPERTS // N_GROUPS
TOPK_GROUPS = 4
ROUTE_SCALE = 2.5

NEG_INF = float("-inf")


def _gate_kernel(x_ref, w_ref, wout_ref, iout_ref):
    x = x_ref[...]
    w = w_ref[...]
    scores = jax.lax.dot_general(
        w, x, (((1,), (1,)), ((), ())), preferred_element_type=jnp.float32
    )
    scores = jax.nn.sigmoid(scores)
    t = scores.shape[1]

    erow = jax.lax.broadcasted_iota(jnp.int32, (N_EXPERTS, t), 0)
    grow8 = jax.lax.broadcasted_iota(jnp.int32, (N_GROUPS, t), 0)

    gmax = jnp.max(scores.reshape(N_GROUPS, GROUP_SIZE, t), axis=1)

    work = gmax
    sel8 = jnp.zeros((N_GROUPS, t), jnp.bool_)
    for _ in range(TOPK_GROUPS):
        m = jnp.max(work, axis=0, keepdims=True)
        cand = jnp.where(work == m, grow8, N_GROUPS)
        best_g = jnp.min(cand, axis=0, keepdims=True)
        pick = grow8 == best_g
        sel8 = jnp.logical_or(sel8, pick)
        work = jnp.where(pick, NEG_INF, work)

    sel = jnp.broadcast_to(sel8[:, None, :], (N_GROUPS, GROUP_SIZE, t)).reshape(
        N_EXPERTS, t
    )
    masked = jnp.where(sel, scores, NEG_INF)
    w_rows = []
    i_rows = []
    for _ in range(TOPK):
        m = jnp.max(masked, axis=0, keepdims=True)
        cand = jnp.where(masked == m, erow, N_EXPERTS)
        best = jnp.min(cand, axis=0, keepdims=True)
        w_rows.append(m)
        i_rows.append(best)
        masked = jnp.where(erow == best, NEG_INF, masked)
    wts = jnp.concatenate(w_rows, axis=0)
    idx = jnp.concatenate(i_rows, axis=0)
    wts = wts / jnp.sum(wts, axis=0, keepdims=True) * ROUTE_SCALE

    wout_ref[...] = wts.T
    iout_ref[...] = idx.T


@functools.partial(jax.jit, static_argnames=())
def kernel(x, weight):
    tokens, dim = x.shape
    tile_t = min(2048, tokens)
    grid = (tokens // tile_t,)
    wts, idx = pl.pallas_call(
        _gate_kernel,
        grid=grid,
        in_specs=[
            pl.BlockSpec((tile_t, dim), lambda i: (i, 0)),
            pl.BlockSpec((N_EXPERTS, dim), lambda i: (0, 0)),
        ],
        out_specs=[
            pl.BlockSpec((tile_t, TOPK), lambda i: (i, 0)),
            pl.BlockSpec((tile_t, TOPK), lambda i: (i, 0)),
        ],
        out_shape=[
            jax.ShapeDtypeStruct((tokens, TOPK), jnp.float32),
            jax.ShapeDtypeStruct((tokens, TOPK), jnp.int32),
        ],
    )(x.astype(jnp.float32), weight.astype(jnp.float32))
    return wts, idx
